# baseline (speedup 1.0000x reference)
import functools

import jax
import jax.numpy as jnp
from jax import lax
from jax.experimental import pallas as pl
from jax.experimental.pallas import tpu as pltpu
from jax.experimental.pallas import tpu_sc as plsc

DEPTH = 8
G = 8
GD = 48
D = G * GD
NW = 32
C = 32


def _sc_body(x_hbm, wd2_hbm, a2_hbm, cv_hbm, v0_hbm,
             xq_hbm, idx_hbm, loss_hbm,
             xbuf, xqbuf, idxbuf, wd2v, a2v, cvv, v0v, lossv):
    wid = lax.axis_index("s") * 2 + lax.axis_index("c")
    ntok = x_hbm.shape[0] // D
    per_w = ntok // NW
    nchunks = per_w // C

    pltpu.sync_copy(wd2_hbm, wd2v)
    pltpu.sync_copy(a2_hbm, a2v)
    pltpu.sync_copy(cv_hbm, cvv)
    pltpu.sync_copy(v0_hbm, v0v)

    iota = lax.iota(jnp.int32, 16)
    zero16 = jnp.zeros((16,), jnp.float32)

    def chunk_body(ci, carry):
        t0 = wid * per_w + ci * C
        pltpu.sync_copy(x_hbm.at[pl.ds(t0 * D, C * D)], xbuf)

        def unit_body(u, carry2):
            tt = u // G
            g = u % G
            rowidx = tt * 16 + iota
            xbase = rowidx * D + g * GD
            s = [zero16] * DEPTH
            for d in range(GD):
                xd = plsc.load_gather(xbuf, [xbase + d])
                for k in range(DEPTH):
                    wv = wd2v[pl.ds(((g * DEPTH + k) * GD + d) * 16, 16)]
                    s[k] = s[k] + xd * wv
            bf = []
            idxf = zero16
            for k in range(DEPTH):
                e = s[k]
                for j in range(k):
                    av = a2v[pl.ds(((j * DEPTH + k) * G + g) * 16, 16)]
                    e = e - bf[j] * av
                cv = cvv[pl.ds((k * G + g) * 16, 16)]
                bit = e > cv
                bf.append(jnp.where(bit, 1.0, 0.0).astype(jnp.float32))
                idxf = idxf + bf[k] * float(1 << k)
            ls = carry2
            for d in range(GD):
                u8 = bf[0] * wd2v[pl.ds(((g * DEPTH + 0) * GD + d) * 16, 16)]
                for k in range(1, DEPTH):
                    u8 = u8 + bf[k] * wd2v[pl.ds(((g * DEPTH + k) * GD + d) * 16, 16)]
                xd = plsc.load_gather(xbuf, [xbase + d])
                xq = 0.5 * u8 + v0v[pl.ds((g * GD + d) * 16, 16)]
                t = xq - xd
                plsc.store_scatter(xqbuf, [xbase + d], xd + t)
                ls = ls + t * t
            plsc.store_scatter(idxbuf, [rowidx * G + g], idxf.astype(jnp.int32))
            return ls

        ls = lax.fori_loop(0, (C // 16) * G, unit_body, carry, unroll=False)
        pltpu.sync_copy(xqbuf, xq_hbm.at[pl.ds(t0 * D, C * D)])
        pltpu.sync_copy(idxbuf, idx_hbm.at[pl.ds(t0 * G, C * G)])
        return ls

    ls = lax.fori_loop(0, nchunks, chunk_body, zero16, unroll=False)
    lossv[...] = ls
    pltpu.sync_copy(lossv, loss_hbm.at[pl.ds(wid * 16, 16)])


def kernel(x, levels):
    B, T, _ = x.shape
    n = B * T
    xf = x.reshape(n * D)

    lv = levels.astype(jnp.float32)
    v0 = lv[:, :, 0, :]
    v1 = lv[:, :, 1, :]
    wd = v1 - v0
    wd2 = 2.0 * wd
    thr0 = jnp.sum(v1 * v1 - v0 * v0, axis=-1)
    p_jk = jnp.einsum('gjd,gkd->gjk', v0, wd)
    jlt = (jnp.arange(DEPTH)[:, None] < jnp.arange(DEPTH)[None, :])
    c = thr0 + 2.0 * jnp.sum(p_jk * jlt[None], axis=1)
    a_jk = 2.0 * jnp.einsum('gjd,gkd->gjk', wd, wd)
    v0sum = jnp.sum(v0, axis=1)

    rep = lambda a: jnp.broadcast_to(a.reshape(-1, 1), (a.size, 16)).reshape(-1)
    wd2f = rep(wd2)
    a2f = rep(jnp.transpose(a_jk, (1, 2, 0)))
    cvf = rep(c.T)
    v0f = rep(v0sum)

    mesh = plsc.VectorSubcoreMesh(core_axis_name="c", subcore_axis_name="s")
    ker = functools.partial(
        pl.kernel, mesh=mesh,
        compiler_params=pltpu.CompilerParams(needs_layout_passes=False),
        out_type=[
            jax.ShapeDtypeStruct((n * D,), jnp.float32),
            jax.ShapeDtypeStruct((n * G,), jnp.int32),
            jax.ShapeDtypeStruct((NW * 16,), jnp.float32),
        ],
        scratch_types=[
            pltpu.VMEM((C * D,), jnp.float32),
            pltpu.VMEM((C * D,), jnp.float32),
            pltpu.VMEM((C * G,), jnp.int32),
            pltpu.VMEM((G * DEPTH * GD * 16,), jnp.float32),
            pltpu.VMEM((DEPTH * DEPTH * G * 16,), jnp.float32),
            pltpu.VMEM((DEPTH * G * 16,), jnp.float32),
            pltpu.VMEM((GD * G * 16,), jnp.float32),
            pltpu.VMEM((16,), jnp.float32),
        ],
    )(_sc_body)
    xq2, idx2, lossp = ker(xf, wd2f, a2f, cvf, v0f)
    total_loss = (2.0 / (B * T * GD)) * jnp.sum(lossp)
    return (xq2.reshape(B, T, D), total_loss, idx2.reshape(B, T, G))


# speedup vs baseline: 19.9127x; 19.9127x over previous
import functools

import jax
import jax.numpy as jnp
from jax.experimental import pallas as pl

DEPTH = 8
G = 8
GD = 48
D = G * GD
GK = G * DEPTH


def _tpq_kernel(x_ref, wd2x_ref, wdt_ref, v0sum_ref, cvec_ref, a2_ref,
                xq_ref, idx_ref, acc_ref, *, blk):
    x = x_ref[...]
    dn = (((1,), (0,)), ((), ()))
    s = jax.lax.dot_general(
        x, wd2x_ref[...], dn,
        precision=jax.lax.Precision.HIGHEST,
        preferred_element_type=jnp.float32)
    st = jnp.transpose(s, (1, 0))
    bitfs = []
    idxf = jnp.zeros((G, blk), jnp.float32)
    for k in range(DEPTH):
        e = st[8 * k:8 * k + 8, :]
        for j in range(k):
            e = e - bitfs[j] * a2_ref[j * DEPTH + k]
        bit = e > cvec_ref[k]
        bf = bit.astype(jnp.float32)
        bitfs.append(bf)
        idxf = idxf + bf * float(1 << k)
    bits64t = jnp.concatenate(bitfs, axis=0)
    bits64 = jnp.transpose(bits64t, (1, 0))
    xq = jax.lax.dot_general(bits64, wdt_ref[...], dn,
                             preferred_element_type=jnp.float32)
    xq = xq + v0sum_ref[...]
    t = xq - x
    xq_ref[...] = x + t
    idx_ref[...] = jnp.transpose(idxf, (1, 0)).astype(jnp.int32)
    p = jnp.sum(t * t)
    i = pl.program_id(0)

    @pl.when(i == 0)
    def _():
        acc_ref[...] = jnp.full((8, 128), p, jnp.float32)

    @pl.when(i > 0)
    def _():
        acc_ref[...] = acc_ref[...] + p


def kernel(x, levels):
    B, T, _ = x.shape
    x2 = x.reshape(B * T, D)
    n = B * T

    lv = levels.astype(jnp.float32)
    v0 = lv[:, :, 0, :]
    v1 = lv[:, :, 1, :]
    wd = v1 - v0
    eye = jnp.eye(G, dtype=jnp.float32)
    wd2x = jnp.einsum('gkd,gh->gdkh', 2.0 * wd, eye).reshape(D, GK)
    wdt = jnp.einsum('gkd,hg->khgd', wd, eye).reshape(GK, D)
    v0sum = jnp.sum(v0, axis=1).reshape(1, D)
    thr0 = jnp.sum(v1 * v1 - v0 * v0, axis=-1)
    p_jk = jnp.einsum('gjd,gkd->gjk', v0, wd)
    jlt = (jnp.arange(DEPTH)[:, None] < jnp.arange(DEPTH)[None, :])
    c = thr0 + 2.0 * jnp.sum(p_jk * jlt[None], axis=1)
    cvec = c.T.reshape(DEPTH, G, 1)
    a_jk = jnp.einsum('gjd,gkd->gjk', wd, wd)
    a2 = 2.0 * jnp.transpose(a_jk, (1, 2, 0)).reshape(DEPTH * DEPTH, G, 1)

    blk = 2048
    grid = n // blk
    xq2, idx2, acc = pl.pallas_call(
        functools.partial(_tpq_kernel, blk=blk),
        grid=(grid,),
        in_specs=[
            pl.BlockSpec((blk, D), lambda i: (i, 0)),
            pl.BlockSpec((D, GK), lambda i: (0, 0)),
            pl.BlockSpec((GK, D), lambda i: (0, 0)),
            pl.BlockSpec((1, D), lambda i: (0, 0)),
            pl.BlockSpec((DEPTH, G, 1), lambda i: (0, 0, 0)),
            pl.BlockSpec((DEPTH * DEPTH, G, 1), lambda i: (0, 0, 0)),
        ],
        out_specs=[
            pl.BlockSpec((blk, D), lambda i: (i, 0)),
            pl.BlockSpec((blk, G), lambda i: (i, 0)),
            pl.BlockSpec((8, 128), lambda i: (0, 0)),
        ],
        out_shape=[
            jax.ShapeDtypeStruct((n, D), jnp.float32),
            jax.ShapeDtypeStruct((n, G), jnp.int32),
            jax.ShapeDtypeStruct((8, 128), jnp.float32),
        ],
    )(x2, wd2x, wdt, v0sum, cvec, a2)

    total_loss = (2.0 / (B * T * GD)) * acc[0, 0]
    return (xq2.reshape(B, T, D), total_loss, idx2.reshape(B, T, G))


# speedup vs baseline: 20.3647x; 1.0227x over previous
import functools

import jax
import jax.numpy as jnp
from jax.experimental import pallas as pl

DEPTH = 8
G = 8
GD = 48
D = G * GD
GK = G * DEPTH


def _tpq_kernel(x_ref, wd2x_ref, wdt_ref, v0sum_ref, cvec_ref, a2_ref,
                xq_ref, idx_ref, acc_ref, *, blk):
    x = x_ref[...]
    dn = (((1,), (0,)), ((), ()))
    s = jax.lax.dot_general(
        x, wd2x_ref[...], dn,
        precision=jax.lax.Precision.HIGHEST,
        preferred_element_type=jnp.float32)
    st = jnp.transpose(s, (1, 0))
    bitfs = []
    idxf = jnp.zeros((G, blk), jnp.float32)
    for k in range(DEPTH):
        e = st[8 * k:8 * k + 8, :]
        for j in range(k):
            e = e - bitfs[j] * a2_ref[j * DEPTH + k]
        bit = e > cvec_ref[k]
        bf = bit.astype(jnp.float32)
        bitfs.append(bf)
        idxf = idxf + bf * float(1 << k)
    bits64t = jnp.concatenate(bitfs, axis=0)
    bits64 = jnp.transpose(bits64t, (1, 0))
    xq = jax.lax.dot_general(bits64, wdt_ref[...], dn,
                             preferred_element_type=jnp.float32)
    xq = xq + v0sum_ref[...]
    t = xq - x
    xq_ref[...] = x + t
    idx_ref[...] = jnp.transpose(idxf, (1, 0)).astype(jnp.int32)
    p = jnp.sum(t * t)
    i = pl.program_id(0)

    @pl.when(i == 0)
    def _():
        acc_ref[...] = jnp.full((8, 128), p, jnp.float32)

    @pl.when(i > 0)
    def _():
        acc_ref[...] = acc_ref[...] + p


def kernel(x, levels):
    B, T, _ = x.shape
    x2 = x.reshape(B * T, D)
    n = B * T

    lv = levels.astype(jnp.float32)
    v0 = lv[:, :, 0, :]
    v1 = lv[:, :, 1, :]
    wd = v1 - v0
    eye = jnp.eye(G, dtype=jnp.float32)
    wd2x = jnp.einsum('gkd,gh->gdkh', 2.0 * wd, eye).reshape(D, GK)
    wdt = jnp.einsum('gkd,hg->khgd', wd, eye).reshape(GK, D)
    v0sum = jnp.sum(v0, axis=1).reshape(1, D)
    thr0 = jnp.sum(v1 * v1 - v0 * v0, axis=-1)
    p_jk = jnp.einsum('gjd,gkd->gjk', v0, wd)
    jlt = (jnp.arange(DEPTH)[:, None] < jnp.arange(DEPTH)[None, :])
    c = thr0 + 2.0 * jnp.sum(p_jk * jlt[None], axis=1)
    cvec = c.T.reshape(DEPTH, G, 1)
    a_jk = jnp.einsum('gjd,gkd->gjk', wd, wd)
    a2 = 2.0 * jnp.transpose(a_jk, (1, 2, 0)).reshape(DEPTH * DEPTH, G, 1)

    blk = 4096
    grid = n // blk
    xq2, idx2, acc = pl.pallas_call(
        functools.partial(_tpq_kernel, blk=blk),
        grid=(grid,),
        in_specs=[
            pl.BlockSpec((blk, D), lambda i: (i, 0)),
            pl.BlockSpec((D, GK), lambda i: (0, 0)),
            pl.BlockSpec((GK, D), lambda i: (0, 0)),
            pl.BlockSpec((1, D), lambda i: (0, 0)),
            pl.BlockSpec((DEPTH, G, 1), lambda i: (0, 0, 0)),
            pl.BlockSpec((DEPTH * DEPTH, G, 1), lambda i: (0, 0, 0)),
        ],
        out_specs=[
            pl.BlockSpec((blk, D), lambda i: (i, 0)),
            pl.BlockSpec((blk, G), lambda i: (i, 0)),
            pl.BlockSpec((8, 128), lambda i: (0, 0)),
        ],
        out_shape=[
            jax.ShapeDtypeStruct((n, D), jnp.float32),
            jax.ShapeDtypeStruct((n, G), jnp.int32),
            jax.ShapeDtypeStruct((8, 128), jnp.float32),
        ],
    )(x2, wd2x, wdt, v0sum, cvec, a2)

    total_loss = (2.0 / (B * T * GD)) * acc[0, 0]
    return (xq2.reshape(B, T, D), total_loss, idx2.reshape(B, T, G))
